# baseline (speedup 1.0000x reference)
import functools
import math

import jax
import jax.numpy as jnp
from jax import lax
from jax.experimental import pallas as pl
from jax.experimental.pallas import tpu as pltpu

_NUM_EXPERTS = 16
_TOPK = 2
_CAP_FACTOR = 1.25
_MIN_CAP = 4


def _cumsum0(m):
    T = m.shape[0]
    row = lax.broadcasted_iota(jnp.int32, m.shape, 0)
    c = m
    sh = 1
    while sh < T:
        rolled = pltpu.roll(c, sh, 0)
        c = c + jnp.where(row >= sh, rolled, 0)
        sh *= 2
    return c


def _route_body(x_ref, w_ref, tp_ref, wfull_ref, stats_ref, counts_ref):
    x = x_ref[...]
    w = w_ref[...]
    logits = lax.dot_general(x, w, (((1,), (1,)), ((), ())),
                             preferred_element_type=jnp.float32)
    T, E = logits.shape
    m = jnp.max(logits, axis=1, keepdims=True)
    p = jnp.exp(logits - m)
    gates = p / jnp.sum(p, axis=1, keepdims=True)

    iota_e = lax.broadcasted_iota(jnp.int32, (T, E), 1)
    m1 = jnp.max(gates, axis=1, keepdims=True)
    idx1 = jnp.min(jnp.where(gates == m1, iota_e, E), axis=1, keepdims=True)
    onehot1 = iota_e == idx1
    g2 = jnp.where(onehot1, -jnp.inf, gates)
    m2 = jnp.max(g2, axis=1, keepdims=True)
    idx2 = jnp.min(jnp.where(g2 == m2, iota_e, E), axis=1, keepdims=True)
    onehot2 = iota_e == idx2

    mask1 = onehot1.astype(jnp.int32)
    mask2 = onehot2.astype(jnp.int32)
    c1 = _cumsum0(mask1)
    c2 = _cumsum0(mask2)
    cnt1 = jnp.sum(mask1, axis=0, keepdims=True)
    cnt2 = jnp.sum(mask2, axis=0, keepdims=True)
    tp = jnp.where(onehot1, c1 - 1,
                   jnp.where(onehot2, cnt1 + c2 - 1, -1))
    tp_ref[...] = tp

    eps = jnp.float32(jnp.finfo(jnp.float32).eps)
    gates_s = jnp.maximum(m1 + m2, eps)
    wfull_ref[...] = gates / gates_s

    capacity = max(int(math.ceil((T / E) * _CAP_FACTOR)), _MIN_CAP)
    expert_capacity = _TOPK * capacity
    valid = jnp.logical_and(tp >= 0, tp < expert_capacity)
    nvalid = jnp.sum(valid.astype(jnp.float32))

    exp_counts = cnt1 + cnt2
    counts_ref[...] = exp_counts
    gsum = jnp.sum(gates, axis=0, keepdims=True)
    l_aux = (jnp.float32(E) * jnp.sum(exp_counts.astype(jnp.float32) * gsum)
             / jnp.float32(T) / jnp.float32(T))
    rate = nvalid / jnp.float32(T * _TOPK)
    stats = jnp.concatenate(
        [jnp.full((1, 4), l_aux, jnp.float32),
         jnp.full((1, 4), rate, jnp.float32)], axis=1)
    stats_ref[...] = stats


def _expand_body(cap, tp_ref, wfull_ref, dispatch_ref, combine_ref):
    tp = tp_ref[...]
    wfull = wfull_ref[...]
    B, E = tp.shape
    c_iota = lax.broadcasted_iota(jnp.int32, (B, E, cap), 2)
    d = c_iota == tp[:, :, None]
    dispatch_ref[...] = d
    combine_ref[...] = jnp.where(d, wfull[:, :, None], jnp.float32(0.0))


def kernel(input, W):
    x = input.astype(jnp.float32).reshape(-1, input.shape[-1])
    T = x.shape[0]
    E = _NUM_EXPERTS
    capacity = max(int(math.ceil((T / E) * _CAP_FACTOR)), _MIN_CAP)
    cap = _TOPK * capacity

    tp, wfull, stats, counts = pl.pallas_call(
        _route_body,
        out_shape=(
            jax.ShapeDtypeStruct((T, E), jnp.int32),
            jax.ShapeDtypeStruct((T, E), jnp.float32),
            jax.ShapeDtypeStruct((1, 8), jnp.float32),
            jax.ShapeDtypeStruct((1, E), jnp.int32),
        ),
    )(x, W)

    BLK = 128
    nblk = T // BLK
    dispatch, combine = pl.pallas_call(
        functools.partial(_expand_body, cap),
        grid=(nblk,),
        in_specs=[
            pl.BlockSpec((BLK, E), lambda i: (i, 0)),
            pl.BlockSpec((BLK, E), lambda i: (i, 0)),
        ],
        out_specs=(
            pl.BlockSpec((BLK, E, cap), lambda i: (i, 0, 0)),
            pl.BlockSpec((BLK, E, cap), lambda i: (i, 0, 0)),
        ),
        out_shape=(
            jax.ShapeDtypeStruct((T, E, cap), jnp.bool_),
            jax.ShapeDtypeStruct((T, E, cap), jnp.float32),
        ),
    )(tp, wfull)

    l_aux = stats[0, 0]
    rate = stats[0, 4]
    z_loss = jnp.zeros((), jnp.float32)
    exp_counts = counts.reshape(E)
    return (l_aux, z_loss, rate, combine, dispatch, exp_counts)


# speedup vs baseline: 3.0553x; 3.0553x over previous
import functools
import math

import jax
import jax.numpy as jnp
from jax import lax
from jax.experimental import pallas as pl
from jax.experimental.pallas import tpu as pltpu

_NUM_EXPERTS = 16
_TOPK = 2
_CAP_FACTOR = 1.25
_MIN_CAP = 4


def _cumsum1(m):
    T = m.shape[1]
    col = lax.broadcasted_iota(jnp.int32, m.shape, 1)
    c = m
    sh = 1
    while sh < T:
        rolled = pltpu.roll(c, sh, 1)
        c = c + jnp.where(col >= sh, rolled, 0)
        sh *= 2
    return c


def _route_body(x_ref, w_ref, tp_ref, wfull_ref, stats_ref, counts_ref):
    x = x_ref[...]
    w = w_ref[...]
    logits = lax.dot_general(w, x, (((1,), (1,)), ((), ())),
                             preferred_element_type=jnp.float32)
    E, T = logits.shape
    m = jnp.max(logits, axis=0, keepdims=True)
    p = jnp.exp(logits - m)
    gates = p / jnp.sum(p, axis=0, keepdims=True)

    iota_e = lax.broadcasted_iota(jnp.int32, (E, T), 0)
    m1 = jnp.max(gates, axis=0, keepdims=True)
    idx1 = jnp.min(jnp.where(gates == m1, iota_e, E), axis=0, keepdims=True)
    onehot1 = iota_e == idx1
    g2 = jnp.where(onehot1, -jnp.inf, gates)
    m2 = jnp.max(g2, axis=0, keepdims=True)
    idx2 = jnp.min(jnp.where(g2 == m2, iota_e, E), axis=0, keepdims=True)
    onehot2 = iota_e == idx2

    mask1 = onehot1.astype(jnp.int32)
    mask2 = onehot2.astype(jnp.int32)
    c1 = _cumsum1(mask1)
    c2 = _cumsum1(mask2)
    cnt1 = jnp.sum(mask1, axis=1, keepdims=True)
    cnt2 = jnp.sum(mask2, axis=1, keepdims=True)
    tp = jnp.where(onehot1, c1 - 1,
                   jnp.where(onehot2, cnt1 + c2 - 1, -1))
    tp_ref[...] = tp

    eps = jnp.float32(jnp.finfo(jnp.float32).eps)
    gates_s = jnp.maximum(m1 + m2, eps)
    wfull_ref[...] = gates / gates_s

    capacity = max(int(math.ceil((T / E) * _CAP_FACTOR)), _MIN_CAP)
    expert_capacity = _TOPK * capacity
    valid = jnp.logical_and(tp >= 0, tp < expert_capacity)
    nvalid = jnp.sum(valid.astype(jnp.float32))

    exp_counts = cnt1 + cnt2
    counts_ref[...] = exp_counts
    gsum = jnp.sum(gates, axis=1, keepdims=True)
    l_aux = (jnp.float32(E) * jnp.sum(exp_counts.astype(jnp.float32) * gsum)
             / jnp.float32(T) / jnp.float32(T))
    rate = nvalid / jnp.float32(T * _TOPK)
    stats = jnp.concatenate(
        [jnp.full((1, 4), l_aux, jnp.float32),
         jnp.full((1, 4), rate, jnp.float32)], axis=1)
    stats_ref[...] = stats


def _expand_body(cap, tp_ref, wfull_ref, dispatch_ref, combine_ref):
    tp = tp_ref[...]
    wfull = wfull_ref[...]
    E, BT = tp.shape
    c_iota = lax.broadcasted_iota(jnp.int32, (E, cap, BT), 1)
    d = c_iota == tp[:, None, :]
    dispatch_ref[...] = d.astype(jnp.int8)
    combine_ref[...] = jnp.where(d, wfull[:, None, :], jnp.float32(0.0))


def kernel(input, W):
    x = input.astype(jnp.float32).reshape(-1, input.shape[-1])
    T = x.shape[0]
    E = _NUM_EXPERTS
    capacity = max(int(math.ceil((T / E) * _CAP_FACTOR)), _MIN_CAP)
    cap = _TOPK * capacity

    tp, wfull, stats, counts = pl.pallas_call(
        _route_body,
        out_shape=(
            jax.ShapeDtypeStruct((E, T), jnp.int32),
            jax.ShapeDtypeStruct((E, T), jnp.float32),
            jax.ShapeDtypeStruct((1, 8), jnp.float32),
            jax.ShapeDtypeStruct((E, 1), jnp.int32),
        ),
    )(x, W)

    BT = 256
    nblk = T // BT
    dispatch_t, combine_t = pl.pallas_call(
        functools.partial(_expand_body, cap),
        grid=(nblk,),
        in_specs=[
            pl.BlockSpec((E, BT), lambda i: (0, i)),
            pl.BlockSpec((E, BT), lambda i: (0, i)),
        ],
        out_specs=(
            pl.BlockSpec((E, cap, BT), lambda i: (0, 0, i)),
            pl.BlockSpec((E, cap, BT), lambda i: (0, 0, i)),
        ),
        out_shape=(
            jax.ShapeDtypeStruct((E, cap, T), jnp.int8),
            jax.ShapeDtypeStruct((E, cap, T), jnp.float32),
        ),
    )(tp, wfull)

    combine = jnp.transpose(combine_t, (2, 0, 1))
    dispatch = jnp.transpose(dispatch_t, (2, 0, 1)).astype(jnp.bool_)
    l_aux = stats[0, 0]
    rate = stats[0, 4]
    z_loss = jnp.zeros((), jnp.float32)
    exp_counts = counts.reshape(E)
    return (l_aux, z_loss, rate, combine, dispatch, exp_counts)
